# baseline (device time: 13974 ns/iter reference)
import jax
import jax.numpy as jnp
from jax import lax
from jax.experimental import pallas as pl
from jax.experimental.pallas import tpu as pltpu

N_DEV = 4
M = 512
N = 512
CHUNK = M // N_DEV
SUB = 4
SROWS = CHUNK // SUB


def kernel(x):
    def body(x_hbm, out_hbm, x_vm, x_bf, rs_buf, ag_buf, out_vm,
             in_sem, out_sems, rs_send, rs_recv, ag_send, ag_recv):
        my = lax.axis_index("i")

        in_copy = pltpu.make_async_copy(x_hbm, x_vm, in_sem)
        in_copy.start()

        barrier_sem = pltpu.get_barrier_semaphore()
        for d in range(1, N_DEV):
            pl.semaphore_signal(
                barrier_sem, inc=1,
                device_id=((my + d) % N_DEV,),
                device_id_type=pl.DeviceIdType.MESH,
            )

        in_copy.wait()
        for d in range(1, N_DEV):
            tgt = (my + d) % N_DEV
            x_bf[pl.ds(tgt * CHUNK, CHUNK), :] = (
                x_vm[pl.ds(tgt * CHUNK, CHUNK), :].astype(jnp.bfloat16)
            )

        pl.semaphore_wait(barrier_sem, N_DEV - 1)

        for s in range(SUB):
            for d in range(1, N_DEV):
                tgt = (my + d) % N_DEV
                pltpu.make_async_remote_copy(
                    src_ref=x_bf.at[pl.ds(tgt * CHUNK + s * SROWS, SROWS), :],
                    dst_ref=rs_buf.at[my, pl.ds(s * SROWS, SROWS), :],
                    send_sem=rs_send.at[d - 1, s],
                    recv_sem=rs_recv.at[d - 1, s],
                    device_id=(tgt,),
                    device_id_type=pl.DeviceIdType.MESH,
                ).start()

        for s in range(SUB):
            for d in range(1, N_DEV):
                src = (my - d) % N_DEV
                pltpu.make_async_remote_copy(
                    src_ref=x_bf.at[pl.ds(0, SROWS), :],
                    dst_ref=rs_buf.at[src, pl.ds(s * SROWS, SROWS), :],
                    send_sem=rs_send.at[d - 1, s],
                    recv_sem=rs_recv.at[d - 1, s],
                    device_id=(src,),
                    device_id_type=pl.DeviceIdType.MESH,
                ).wait_recv()

            sl = pl.ds(s * SROWS, SROWS)
            rows = pl.ds(my * CHUNK + s * SROWS, SROWS)
            acc = x_vm[rows, :]
            for d in range(1, N_DEV):
                src = (my - d) % N_DEV
                acc = acc + rs_buf[src, sl, :].astype(jnp.float32)
            out_vm[rows, :] = acc
            ag_buf[my, sl, :] = acc.astype(jnp.bfloat16)
            pltpu.make_async_copy(
                out_vm.at[rows, :], out_hbm.at[rows, :], out_sems.at[0, s]
            ).start()

            for d in range(1, N_DEV):
                tgt = (my + d) % N_DEV
                pltpu.make_async_remote_copy(
                    src_ref=ag_buf.at[my, sl, :],
                    dst_ref=ag_buf.at[my, sl, :],
                    send_sem=ag_send.at[d - 1, s],
                    recv_sem=ag_recv.at[d - 1, s],
                    device_id=(tgt,),
                    device_id_type=pl.DeviceIdType.MESH,
                ).start()

        for s in range(SUB):
            sl = pl.ds(s * SROWS, SROWS)
            for d in range(1, N_DEV):
                src = (my - d) % N_DEV
                pltpu.make_async_remote_copy(
                    src_ref=x_bf.at[pl.ds(0, SROWS), :],
                    dst_ref=ag_buf.at[src, sl, :],
                    send_sem=ag_send.at[d - 1, s],
                    recv_sem=ag_recv.at[d - 1, s],
                    device_id=(src,),
                    device_id_type=pl.DeviceIdType.MESH,
                ).wait_recv()
                rows = pl.ds(src * CHUNK + s * SROWS, SROWS)
                out_vm[rows, :] = ag_buf[src, sl, :].astype(jnp.float32)
                pltpu.make_async_copy(
                    out_vm.at[rows, :], out_hbm.at[rows, :], out_sems.at[d, s]
                ).start()

        for s in range(SUB):
            sl = pl.ds(s * SROWS, SROWS)
            rows_own = pl.ds(my * CHUNK + s * SROWS, SROWS)
            pltpu.make_async_copy(
                out_vm.at[rows_own, :], out_hbm.at[rows_own, :], out_sems.at[0, s]
            ).wait()
            for d in range(1, N_DEV):
                src = (my - d) % N_DEV
                rows = pl.ds(src * CHUNK + s * SROWS, SROWS)
                pltpu.make_async_copy(
                    out_vm.at[rows, :], out_hbm.at[rows, :], out_sems.at[d, s]
                ).wait()
                tgt = (my + d) % N_DEV
                pltpu.make_async_remote_copy(
                    src_ref=x_bf.at[pl.ds(tgt * CHUNK + s * SROWS, SROWS), :],
                    dst_ref=rs_buf.at[my, sl, :],
                    send_sem=rs_send.at[d - 1, s],
                    recv_sem=rs_recv.at[d - 1, s],
                    device_id=(tgt,),
                    device_id_type=pl.DeviceIdType.MESH,
                ).wait_send()
                pltpu.make_async_remote_copy(
                    src_ref=ag_buf.at[my, sl, :],
                    dst_ref=ag_buf.at[my, sl, :],
                    send_sem=ag_send.at[d - 1, s],
                    recv_sem=ag_recv.at[d - 1, s],
                    device_id=(tgt,),
                    device_id_type=pl.DeviceIdType.MESH,
                ).wait_send()

    return pl.pallas_call(
        body,
        out_shape=jax.ShapeDtypeStruct((M, N), jnp.float32),
        in_specs=[pl.BlockSpec(memory_space=pltpu.MemorySpace.HBM)],
        out_specs=pl.BlockSpec(memory_space=pltpu.MemorySpace.HBM),
        scratch_shapes=[
            pltpu.VMEM((M, N), jnp.float32),
            pltpu.VMEM((M, N), jnp.bfloat16),
            pltpu.VMEM((N_DEV, CHUNK, N), jnp.bfloat16),
            pltpu.VMEM((N_DEV, CHUNK, N), jnp.bfloat16),
            pltpu.VMEM((M, N), jnp.float32),
            pltpu.SemaphoreType.DMA,
            pltpu.SemaphoreType.DMA((N_DEV, SUB)),
            pltpu.SemaphoreType.DMA((N_DEV - 1, SUB)),
            pltpu.SemaphoreType.DMA((N_DEV - 1, SUB)),
            pltpu.SemaphoreType.DMA((N_DEV - 1, SUB)),
            pltpu.SemaphoreType.DMA((N_DEV - 1, SUB)),
        ],
        compiler_params=pltpu.CompilerParams(collective_id=0),
    )(x)
